# Initial kernel scaffold; baseline (speedup 1.0000x reference)
import jax
import jax.numpy as jnp
from jax.experimental import pallas as pl


def kernel(h, x, batch_num_nodes, a):
    raise NotImplementedError("write your pallas kernel here")


# speedup vs baseline: 16.2696x; 16.2696x over previous
import functools

import jax
import jax.numpy as jnp
from jax.experimental import pallas as pl
from jax.experimental.pallas import tpu as pltpu

_B = 1536
_W = 64


def _attn_body(s0_ref, x_ref, lo_r_ref, hi_r_ref, lo_c_ref, hi_c_ref,
               h_ref, a_ref, out_ref, hx_s, acc_s, z_s, *, nb, bsz, w):
    b = pl.program_id(0)

    @pl.when(b == 0)
    def _init():
        hx_s[...] = jnp.dot(h_ref[...], a_ref[...],
                            preferred_element_type=jnp.float32)
        acc_s[...] = jnp.zeros_like(acc_s)
        z_s[...] = jnp.zeros_like(z_s)

    s0 = s0_ref[b]
    xb = x_ref[...]

    rows_c = b * bsz + jax.lax.broadcasted_iota(jnp.int32, (bsz, 1), 0)
    oh = ((rows_c >= lo_r_ref[0]) & (rows_c < hi_r_ref[0])).astype(jnp.float32)
    rows_r = b * bsz + jax.lax.broadcasted_iota(jnp.int32, (1, bsz), 1)
    oht = ((rows_r >= lo_c_ref[0]) & (rows_r < hi_c_ref[0])).astype(jnp.float32)

    hxw = hx_s[pl.ds(s0, w), :]
    hxg = jnp.dot(oh, hxw, preferred_element_type=jnp.float32)
    e = jnp.sum(xb * hxg, axis=1, keepdims=True)
    ex = jnp.exp(e)
    wx = xb * ex

    acc_s[pl.ds(s0, w), :] += jnp.dot(oht, wx, preferred_element_type=jnp.float32)
    z_s[pl.ds(s0, w), :] += jnp.dot(oht, ex, preferred_element_type=jnp.float32)

    @pl.when(b == nb - 1)
    def _fin():
        z = z_s[...]
        out_ref[...] = jnp.where(z > 0, acc_s[...] / z, 0.0)


@jax.jit
def kernel(h, x, batch_num_nodes, a):
    m, d_h = h.shape
    n, d_x = x.shape
    bsz, w = _B, _W
    assert n % bsz == 0
    nb = n // bsz

    bnn = batch_num_nodes.astype(jnp.int32)
    offsets = jnp.concatenate([jnp.zeros((1,), jnp.int32), jnp.cumsum(bnn)])
    starts = jnp.arange(nb, dtype=jnp.int32) * bsz
    s0 = jnp.searchsorted(offsets, starts, side='right').astype(jnp.int32) - 1
    s0 = jnp.clip(s0 & ~7, 0, m - w)
    widx = s0[:, None] + jnp.arange(w, dtype=jnp.int32)[None, :]
    lo = offsets[jnp.clip(widx, 0, m)]
    hi = offsets[jnp.clip(widx + 1, 0, m)]

    grid_spec = pltpu.PrefetchScalarGridSpec(
        num_scalar_prefetch=1,
        grid=(nb,),
        in_specs=[
            pl.BlockSpec((bsz, d_x), lambda b, s0a: (b, 0)),
            pl.BlockSpec((1, 1, w), lambda b, s0a: (b, 0, 0)),
            pl.BlockSpec((1, 1, w), lambda b, s0a: (b, 0, 0)),
            pl.BlockSpec((1, w, 1), lambda b, s0a: (b, 0, 0)),
            pl.BlockSpec((1, w, 1), lambda b, s0a: (b, 0, 0)),
            pl.BlockSpec((m, d_h), lambda b, s0a: (0, 0)),
            pl.BlockSpec((d_h, d_x), lambda b, s0a: (0, 0)),
        ],
        out_specs=pl.BlockSpec((m, d_x), lambda b, s0a: (0, 0)),
        scratch_shapes=[
            pltpu.VMEM((m, d_x), jnp.float32),
            pltpu.VMEM((m, d_x), jnp.float32),
            pltpu.VMEM((m, 1), jnp.float32),
        ],
    )

    body = functools.partial(_attn_body, nb=nb, bsz=bsz, w=w)
    out = pl.pallas_call(
        body,
        grid_spec=grid_spec,
        out_shape=jax.ShapeDtypeStruct((m, d_x), jnp.float32),
    )(s0, x, lo.reshape(nb, 1, w), hi.reshape(nb, 1, w),
      lo.reshape(nb, w, 1), hi.reshape(nb, w, 1), h, a)
    return out


# speedup vs baseline: 17.8352x; 1.0962x over previous
import functools

import jax
import jax.numpy as jnp
from jax.experimental import pallas as pl
from jax.experimental.pallas import tpu as pltpu

_B = 1536
_W = 64


def _attn_body(s0_ref, x_ref, lo_r_ref, hi_r_ref,
               h_ref, a_ref, out_ref, hx_s, acc_s, z_s, *, nb, bsz, w):
    b = pl.program_id(0)

    @pl.when(b == 0)
    def _init():
        hx_s[...] = jnp.dot(h_ref[...], a_ref[...],
                            preferred_element_type=jnp.float32)
        acc_s[...] = jnp.zeros_like(acc_s)
        z_s[...] = jnp.zeros_like(z_s)

    s0 = s0_ref[b]
    xb = x_ref[...]

    rows_c = b * bsz + jax.lax.broadcasted_iota(jnp.int32, (bsz, 1), 0)
    oh = (rows_c >= lo_r_ref[0]) & (rows_c < hi_r_ref[0])

    hxw = hx_s[pl.ds(s0, w), :]
    scores = jax.lax.dot_general(xb, hxw, (((1,), (1,)), ((), ())),
                                 preferred_element_type=jnp.float32)
    ohe = jnp.where(oh, jnp.exp(scores), 0.0)

    contrib = jax.lax.dot_general(ohe, xb, (((0,), (0,)), ((), ())),
                                  preferred_element_type=jnp.float32)
    zrow = jnp.sum(ohe, axis=0, keepdims=True)

    acc_s[pl.ds(s0, w), :] += contrib
    z_s[pl.ds(s0, w), :] += zrow.T

    @pl.when(b == nb - 1)
    def _fin():
        z = z_s[...]
        out_ref[...] = jnp.where(z > 0, acc_s[...] / z, 0.0)


@jax.jit
def kernel(h, x, batch_num_nodes, a):
    m, d_h = h.shape
    n, d_x = x.shape
    bsz, w = _B, _W
    assert n % bsz == 0
    nb = n // bsz

    bnn = batch_num_nodes.astype(jnp.int32)
    offsets = jnp.concatenate([jnp.zeros((1,), jnp.int32), jnp.cumsum(bnn)])
    starts = jnp.arange(nb, dtype=jnp.int32) * bsz
    s0 = jnp.searchsorted(offsets, starts, side='right').astype(jnp.int32) - 1
    s0 = jnp.clip(s0 & ~7, 0, m - w)
    widx = s0[:, None] + jnp.arange(w, dtype=jnp.int32)[None, :]
    lo = offsets[jnp.clip(widx, 0, m)]
    hi = offsets[jnp.clip(widx + 1, 0, m)]

    grid_spec = pltpu.PrefetchScalarGridSpec(
        num_scalar_prefetch=1,
        grid=(nb,),
        in_specs=[
            pl.BlockSpec((bsz, d_x), lambda b, s0a: (b, 0)),
            pl.BlockSpec((1, 1, w), lambda b, s0a: (b, 0, 0)),
            pl.BlockSpec((1, 1, w), lambda b, s0a: (b, 0, 0)),
            pl.BlockSpec((m, d_h), lambda b, s0a: (0, 0)),
            pl.BlockSpec((d_h, d_x), lambda b, s0a: (0, 0)),
        ],
        out_specs=pl.BlockSpec((m, d_x), lambda b, s0a: (0, 0)),
        scratch_shapes=[
            pltpu.VMEM((m, d_x), jnp.float32),
            pltpu.VMEM((m, d_x), jnp.float32),
            pltpu.VMEM((m, 1), jnp.float32),
        ],
    )

    body = functools.partial(_attn_body, nb=nb, bsz=bsz, w=w)
    out = pl.pallas_call(
        body,
        grid_spec=grid_spec,
        out_shape=jax.ShapeDtypeStruct((m, d_x), jnp.float32),
    )(s0, x, lo.reshape(nb, 1, w), hi.reshape(nb, 1, w), h, a)
    return out
